# baseline (speedup 1.0000x reference)
import functools

import jax
import jax.numpy as jnp
from jax import lax
from jax.experimental import pallas as pl
from jax.experimental.pallas import tpu as pltpu
from jax.experimental.pallas import tpu_sc as plsc

_VOCAB = 1000
_D = 32
_B = 16384
_H = 200
_F = 64
_HID = 128

_NC = 2
_NS = 16
_NW = _NC * _NS
_RPW = _B // _NW
_BLK = 16
_NBLK = _RPW // _BLK
_WPR = 8
_OW = 16

_PERM = ([4 * j for j in range(8)] + [4 * j + 1 for j in range(8)]
         + [4 * j + 2 for j in range(8)] + [4 * j + 3 for j in range(8)])

_GDN = lax.GatherDimensionNumbers(
    offset_dims=(), collapsed_slice_dims=(0,), start_index_map=(0,))


def _vperm(vec, pat):
    return lax.gather(vec, pat, _GDN, slice_sizes=(1,),
                      mode=lax.GatherScatterMode.PROMISE_IN_BOUNDS)


def _sc_embed_sum(word_inputs, tbl_q):
    mesh = plsc.VectorSubcoreMesh(
        core_axis_name="c", subcore_axis_name="s",
        num_cores=_NC, num_subcores=_NS)

    cp = pltpu.CompilerParams(
        needs_layout_passes=False, use_tc_tiling_on_sc=False)

    @functools.partial(
        pl.kernel,
        out_type=jax.ShapeDtypeStruct((_B, 128), jnp.int32),
        mesh=mesh,
        compiler_params=cp,
        scratch_types=[
            pltpu.VMEM((_VOCAB * _WPR,), jnp.int32),
            pltpu.VMEM((_BLK, 256), jnp.int32),
            pltpu.VMEM((_BLK, 256), jnp.int32),
            pltpu.VMEM((_BLK, _OW), jnp.int32),
            pltpu.SemaphoreType.DMA,
            pltpu.SemaphoreType.DMA,
        ],
    )
    def k(word_hbm, tbl_hbm, out_hbm, tbl_v, idx0_v, idx1_v, res_v,
          sem0, sem1):
        wid = lax.axis_index("s") * _NC + lax.axis_index("c")
        lane = lax.iota(jnp.int32, 16)
        loff = lane & 7
        mask = jnp.int32(0x00FF00FF)
        half = lane >> 3
        pats = [jnp.reshape(half + 2 * p, (16, 1)) for p in range(8)]
        fold_pat = jnp.reshape(loff + 8, (16, 1))
        low_pat = jnp.reshape(loff, (16, 1))
        blk0 = wid * _NBLK

        def start_in(blk, idx_v, sem):
            base = (blk0 + blk) * _BLK
            pltpu.async_copy(word_hbm.at[pl.ds(base, _BLK), :], idx_v, sem)

        def wait_in(idx_v, sem):
            pltpu.make_async_copy(word_hbm.at[pl.ds(0, _BLK), :],
                                  idx_v, sem).wait()

        def compute(blk, idx_v):
            @pl.loop(0, _BLK)
            def _(r):
                def do_chunk(cbase, accs, pairs):
                    chunk = idx_v[r, pl.ds(cbase, 16)] << 3
                    for p in pairs:
                        addr = _vperm(chunk, pats[p]) + loff
                        w = plsc.load_gather(tbl_v, [addr])
                        lo = w & mask
                        hi = lax.shift_right_logical(w, 8) & mask
                        accs[p % 2] = accs[p % 2] + lo
                        accs[2 + p % 2] = accs[2 + p % 2] + hi
                    return accs

                def h_body(i, carry):
                    return tuple(do_chunk(i * 16, list(carry), range(8)))

                zero = jnp.zeros((16,), jnp.int32)
                accs = lax.fori_loop(0, _H // 16, h_body, (zero,) * 4)
                accs = do_chunk(_H - 16, list(accs), range(4, 8))
                acc_lo = accs[0] + accs[1]
                acc_hi = accs[2] + accs[3]
                lo_f = acc_lo + _vperm(acc_lo, fold_pat)
                hi_f = acc_hi + _vperm(acc_hi, fold_pat)
                res_v[r, :] = jnp.where(lane < 8, lo_f, _vperm(hi_f, low_pat))

            base = (blk0 + blk) * _BLK
            pltpu.sync_copy(res_v, out_hbm.at[pl.ds(base, _BLK), pl.ds(0, _OW)])

        pltpu.sync_copy(tbl_hbm, tbl_v)
        start_in(0, idx0_v, sem0)
        start_in(1, idx1_v, sem1)

        @pl.loop(0, _NBLK, step=2)
        def _(blk):
            wait_in(idx0_v, sem0)
            compute(blk, idx0_v)

            @pl.when(blk + 2 < _NBLK)
            def _():
                start_in(blk + 2, idx0_v, sem0)

            wait_in(idx1_v, sem1)
            compute(blk + 1, idx1_v)

            @pl.when(blk + 3 < _NBLK)
            def _():
                start_in(blk + 3, idx1_v, sem1)

    return k(jnp.pad(word_inputs, ((0, 0), (0, 256 - _H))), tbl_q)


def _tc_mlp(emb_packed, feats, W1p, b1, W2, b2, scale):
    MT = 2048

    def body(emb_ref, feat_ref, w1_ref, b1_ref, w2_ref, b2_ref, sc_ref,
             out_ref):
        u = emb_ref[:, 0:_OW]
        lo = (u & jnp.int32(0xFFFF)).astype(jnp.float32)
        hi = lax.shift_right_logical(u, 16).astype(jnp.float32)
        sums = jnp.concatenate([lo, hi], axis=1)
        sc = sc_ref[0, 0]
        w1e = w1_ref[0:_D, :]
        w1f = w1_ref[_D:, :]
        h = jnp.dot(sums, w1e, preferred_element_type=jnp.float32) * (sc / _H)
        h = h + jnp.dot(feat_ref[...], w1f, preferred_element_type=jnp.float32)
        b1eff = b1_ref[...] - (128.0 * sc) * jnp.sum(w1e, 0, keepdims=True)
        h = jnp.maximum(h + b1eff, 0.0)
        out_ref[...] = (jnp.dot(h, w2_ref[...],
                                preferred_element_type=jnp.float32)
                        + b2_ref[...])

    nout = b2.shape[-1]
    return pl.pallas_call(
        body,
        grid=(_B // MT,),
        in_specs=[
            pl.BlockSpec((MT, 128), lambda i: (i, 0)),
            pl.BlockSpec((MT, _F), lambda i: (i, 0)),
            pl.BlockSpec((_D + _F, _HID), lambda i: (0, 0)),
            pl.BlockSpec((1, _HID), lambda i: (0, 0)),
            pl.BlockSpec((_HID, nout), lambda i: (0, 0)),
            pl.BlockSpec((1, nout), lambda i: (0, 0)),
            pl.BlockSpec((1, 1), lambda i: (0, 0)),
        ],
        out_specs=pl.BlockSpec((MT, nout), lambda i: (i, 0)),
        out_shape=jax.ShapeDtypeStruct((_B, nout), jnp.float32),
    )(emb_packed, feats, W1p, b1, W2, b2, scale)


def kernel(word_inputs, feature_inputs, emb_table, W1, b1, W2, b2):
    scale = jnp.maximum(jnp.max(jnp.abs(emb_table)), 1e-30) / 127.0
    q = jnp.clip(jnp.round(emb_table / scale) + 128.0, 0.0, 255.0)
    q = q.astype(jnp.int32).reshape(_VOCAB, _WPR, 4)
    tbl_q = (q[..., 0] | (q[..., 1] << 8) | (q[..., 2] << 16)
             | (q[..., 3] << 24)).reshape(_VOCAB * _WPR)

    emb_packed = _sc_embed_sum(word_inputs.astype(jnp.int32), tbl_q)

    perm = jnp.array(_PERM, jnp.int32)
    W1p = jnp.concatenate([W1[:_D][perm], W1[_D:]], axis=0)
    out = _tc_mlp(emb_packed, feature_inputs.astype(jnp.float32), W1p,
                  b1.reshape(1, -1), W2, b2.reshape(1, -1),
                  scale.reshape(1, 1).astype(jnp.float32))
    return out.astype(jnp.float64)


# speedup vs baseline: 1.0336x; 1.0336x over previous
import functools

import jax
import jax.numpy as jnp
from jax import lax
from jax.experimental import pallas as pl
from jax.experimental.pallas import tpu as pltpu
from jax.experimental.pallas import tpu_sc as plsc

_VOCAB = 1000
_D = 32
_B = 16384
_H = 200
_F = 64
_HID = 128

_NC = 2
_NS = 16
_NW = _NC * _NS
_RPW = _B // _NW
_BLK = 16
_NBLK = _RPW // _BLK
_WPR = 8
_OW = 16

_PERM = ([4 * j for j in range(8)] + [4 * j + 1 for j in range(8)]
         + [4 * j + 2 for j in range(8)] + [4 * j + 3 for j in range(8)])

_GDN = lax.GatherDimensionNumbers(
    offset_dims=(), collapsed_slice_dims=(0,), start_index_map=(0,))


def _vperm(vec, pat):
    return lax.gather(vec, pat, _GDN, slice_sizes=(1,),
                      mode=lax.GatherScatterMode.PROMISE_IN_BOUNDS)


def _sc_embed_sum(word_inputs, tbl_q):
    mesh = plsc.VectorSubcoreMesh(
        core_axis_name="c", subcore_axis_name="s",
        num_cores=_NC, num_subcores=_NS)

    cp = pltpu.CompilerParams(
        needs_layout_passes=False, use_tc_tiling_on_sc=False)

    @functools.partial(
        pl.kernel,
        out_type=jax.ShapeDtypeStruct((_B, 128), jnp.int32),
        mesh=mesh,
        compiler_params=cp,
        scratch_types=[
            pltpu.VMEM((_VOCAB * _WPR,), jnp.int32),
            pltpu.VMEM((_BLK, _H), jnp.int32),
            pltpu.VMEM((_BLK, _H), jnp.int32),
            pltpu.VMEM((_BLK, _OW), jnp.int32),
            pltpu.SemaphoreType.DMA,
            pltpu.SemaphoreType.DMA,
        ],
    )
    def k(word_hbm, tbl_hbm, out_hbm, tbl_v, idx0_v, idx1_v, res_v,
          sem0, sem1):
        wid = lax.axis_index("s") * _NC + lax.axis_index("c")
        lane = lax.iota(jnp.int32, 16)
        loff = lane & 7
        mask = jnp.int32(0x00FF00FF)
        half = lane >> 3
        pats = [jnp.reshape(half + 2 * p, (16, 1)) for p in range(8)]
        fold_pat = jnp.reshape(loff + 8, (16, 1))
        low_pat = jnp.reshape(loff, (16, 1))
        blk0 = wid * _NBLK

        def start_in(blk, idx_v, sem):
            base = (blk0 + blk) * _BLK
            pltpu.async_copy(word_hbm.at[pl.ds(base, _BLK), :], idx_v, sem)

        def wait_in(idx_v, sem):
            pltpu.make_async_copy(word_hbm.at[pl.ds(0, _BLK), :],
                                  idx_v, sem).wait()

        def compute(blk, idx_v):
            @pl.loop(0, _BLK)
            def _(r):
                def do_chunk(cbase, accs, pairs):
                    chunk = idx_v[r, pl.ds(cbase, 16)] << 3
                    for p in pairs:
                        addr = _vperm(chunk, pats[p]) + loff
                        w = plsc.load_gather(tbl_v, [addr])
                        lo = w & mask
                        hi = lax.shift_right_logical(w, 8) & mask
                        accs[p % 2] = accs[p % 2] + lo
                        accs[2 + p % 2] = accs[2 + p % 2] + hi
                    return accs

                def h_body(i, carry):
                    return tuple(do_chunk(i * 16, list(carry), range(8)))

                zero = jnp.zeros((16,), jnp.int32)
                accs = lax.fori_loop(0, _H // 16, h_body, (zero,) * 4)
                accs = do_chunk(_H - 16, list(accs), range(4, 8))
                acc_lo = accs[0] + accs[1]
                acc_hi = accs[2] + accs[3]
                lo_f = acc_lo + _vperm(acc_lo, fold_pat)
                hi_f = acc_hi + _vperm(acc_hi, fold_pat)
                res_v[r, :] = jnp.where(lane < 8, lo_f, _vperm(hi_f, low_pat))

            base = (blk0 + blk) * _BLK
            pltpu.sync_copy(res_v, out_hbm.at[pl.ds(base, _BLK), pl.ds(0, _OW)])

        pltpu.sync_copy(tbl_hbm, tbl_v)
        start_in(0, idx0_v, sem0)
        start_in(1, idx1_v, sem1)

        @pl.loop(0, _NBLK, step=2)
        def _(blk):
            wait_in(idx0_v, sem0)
            compute(blk, idx0_v)

            @pl.when(blk + 2 < _NBLK)
            def _():
                start_in(blk + 2, idx0_v, sem0)

            wait_in(idx1_v, sem1)
            compute(blk + 1, idx1_v)

            @pl.when(blk + 3 < _NBLK)
            def _():
                start_in(blk + 3, idx1_v, sem1)

    return k(word_inputs, tbl_q)


def _tc_mlp(emb_packed, feats, W1p, b1, W2, b2, scale):
    MT = 2048

    def body(emb_ref, feat_ref, w1_ref, b1_ref, w2_ref, b2_ref, sc_ref,
             out_ref):
        u = emb_ref[:, 0:_OW]
        lo = (u & jnp.int32(0xFFFF)).astype(jnp.float32)
        hi = lax.shift_right_logical(u, 16).astype(jnp.float32)
        sums = jnp.concatenate([lo, hi], axis=1)
        sc = sc_ref[0, 0]
        w1e = w1_ref[0:_D, :]
        w1f = w1_ref[_D:, :]
        h = jnp.dot(sums, w1e, preferred_element_type=jnp.float32) * (sc / _H)
        h = h + jnp.dot(feat_ref[...], w1f, preferred_element_type=jnp.float32)
        b1eff = b1_ref[...] - (128.0 * sc) * jnp.sum(w1e, 0, keepdims=True)
        h = jnp.maximum(h + b1eff, 0.0)
        out_ref[...] = (jnp.dot(h, w2_ref[...],
                                preferred_element_type=jnp.float32)
                        + b2_ref[...])

    nout = b2.shape[-1]
    return pl.pallas_call(
        body,
        grid=(_B // MT,),
        in_specs=[
            pl.BlockSpec((MT, 128), lambda i: (i, 0)),
            pl.BlockSpec((MT, _F), lambda i: (i, 0)),
            pl.BlockSpec((_D + _F, _HID), lambda i: (0, 0)),
            pl.BlockSpec((1, _HID), lambda i: (0, 0)),
            pl.BlockSpec((_HID, nout), lambda i: (0, 0)),
            pl.BlockSpec((1, nout), lambda i: (0, 0)),
            pl.BlockSpec((1, 1), lambda i: (0, 0)),
        ],
        out_specs=pl.BlockSpec((MT, nout), lambda i: (i, 0)),
        out_shape=jax.ShapeDtypeStruct((_B, nout), jnp.float32),
    )(emb_packed, feats, W1p, b1, W2, b2, scale)


def kernel(word_inputs, feature_inputs, emb_table, W1, b1, W2, b2):
    scale = jnp.maximum(jnp.max(jnp.abs(emb_table)), 1e-30) / 127.0
    q = jnp.clip(jnp.round(emb_table / scale) + 128.0, 0.0, 255.0)
    q = q.astype(jnp.int32).reshape(_VOCAB, _WPR, 4)
    tbl_q = (q[..., 0] | (q[..., 1] << 8) | (q[..., 2] << 16)
             | (q[..., 3] << 24)).reshape(_VOCAB * _WPR)

    emb_packed = _sc_embed_sum(word_inputs.astype(jnp.int32), tbl_q)

    perm = jnp.array(_PERM, jnp.int32)
    W1p = jnp.concatenate([W1[:_D][perm], W1[_D:]], axis=0)
    out = _tc_mlp(emb_packed, feature_inputs.astype(jnp.float32), W1p,
                  b1.reshape(1, -1), W2, b2.reshape(1, -1),
                  scale.reshape(1, 1).astype(jnp.float32))
    return out.astype(jnp.float64)
